# initial kernel scaffold (unmeasured)
import jax
import jax.numpy as jnp
from jax import lax
from jax.experimental import pallas as pl
from jax.experimental.pallas import tpu as pltpu

N_DEV = 16
CHUNK = 64


def kernel(x, dy):
    k, m = x.shape
    _, f = dy.shape
    n_hops = N_DEV - 1

    def body(x_ref, dy_ref, out_ref, acc_ref, comm_ref, send_sems, recv_sems):
        me = lax.axis_index("i")
        left = lax.rem(me + N_DEV - 1, N_DEV)
        right = lax.rem(me + 1, N_DEV)

        barrier_sem = pltpu.get_barrier_semaphore()
        for nbr in (left, right):
            pl.semaphore_signal(
                barrier_sem, inc=1,
                device_id=(nbr,), device_id_type=pl.DeviceIdType.MESH,
            )
        pl.semaphore_wait(barrier_sem, 2)

        acc_ref[:, :] = lax.dot_general(
            x_ref[:, :], dy_ref[:, :],
            dimension_numbers=(((0,), (0,)), ((), ())),
            preferred_element_type=jnp.float32,
        )

        for h in range(n_hops):
            c_send = lax.rem(me + 2 * N_DEV - 1 - h, N_DEV)
            rdma = pltpu.make_async_remote_copy(
                src_ref=acc_ref.at[pl.ds(c_send * CHUNK, CHUNK), :],
                dst_ref=comm_ref.at[h],
                send_sem=send_sems.at[h],
                recv_sem=recv_sems.at[h],
                device_id=(right,),
                device_id_type=pl.DeviceIdType.MESH,
            )
            rdma.start()
            rdma.wait()
            c_recv = lax.rem(me + 2 * N_DEV - 2 - h, N_DEV)
            sl = pl.ds(c_recv * CHUNK, CHUNK)
            acc_ref[sl, :] = acc_ref[sl, :] + comm_ref[h]

        out_ref[:, :] = acc_ref[pl.ds(me * CHUNK, CHUNK), :]

    return pl.pallas_call(
        body,
        out_shape=jax.ShapeDtypeStruct((CHUNK, f), jnp.float32),
        in_specs=[
            pl.BlockSpec(memory_space=pltpu.VMEM),
            pl.BlockSpec(memory_space=pltpu.VMEM),
        ],
        out_specs=pl.BlockSpec(memory_space=pltpu.VMEM),
        scratch_shapes=[
            pltpu.VMEM((N_DEV * CHUNK, f), jnp.float32),
            pltpu.VMEM((n_hops, CHUNK, f), jnp.float32),
            pltpu.SemaphoreType.DMA((n_hops,)),
            pltpu.SemaphoreType.DMA((n_hops,)),
        ],
        compiler_params=pltpu.CompilerParams(collective_id=0),
    )(x, dy)


# baseline (device time: 220944 ns/iter reference)
import jax
import jax.numpy as jnp
from jax import lax
from jax.experimental import pallas as pl
from jax.experimental.pallas import tpu as pltpu

N_DEV = 16
CHUNK = 64


def kernel(x, dy):
    k, m = x.shape
    _, f = dy.shape
    n_hops = N_DEV - 1

    def body(x_ref, dy_ref, out_ref, acc_ref, comm_ref, send_sems, recv_sems):
        me = lax.axis_index("i")
        left = lax.rem(me + N_DEV - 1, N_DEV)
        right = lax.rem(me + 1, N_DEV)

        barrier_sem = pltpu.get_barrier_semaphore()
        for nbr in (left, right):
            pl.semaphore_signal(
                barrier_sem, inc=1,
                device_id=(nbr,), device_id_type=pl.DeviceIdType.MESH,
            )
        pl.semaphore_wait(barrier_sem, 2)

        acc_ref[:, :] = lax.dot_general(
            x_ref[:, :], dy_ref[:, :],
            dimension_numbers=(((0,), (0,)), ((), ())),
            preferred_element_type=jnp.float32,
        )

        for h in range(n_hops):
            c_send = lax.rem(me + 2 * N_DEV - 1 - h, N_DEV)
            rdma = pltpu.make_async_remote_copy(
                src_ref=acc_ref.at[pl.ds(c_send * CHUNK, CHUNK), :],
                dst_ref=comm_ref.at[h],
                send_sem=send_sems.at[h],
                recv_sem=recv_sems.at[h],
                device_id=(right,),
                device_id_type=pl.DeviceIdType.MESH,
            )
            rdma.start()
            rdma.wait()
            c_recv = lax.rem(me + 2 * N_DEV - 2 - h, N_DEV)
            sl = pl.ds(c_recv * CHUNK, CHUNK)
            acc_ref[sl, :] = acc_ref[sl, :] + comm_ref[h]

        out_ref[:, :] = acc_ref[pl.ds(me * CHUNK, CHUNK), :]

    return pl.pallas_call(
        body,
        out_shape=jax.ShapeDtypeStruct((CHUNK, f), jnp.float32),
        in_specs=[
            pl.BlockSpec(memory_space=pltpu.VMEM),
            pl.BlockSpec(memory_space=pltpu.VMEM),
        ],
        out_specs=pl.BlockSpec(memory_space=pltpu.VMEM),
        scratch_shapes=[
            pltpu.VMEM((N_DEV * CHUNK, f), jnp.float32),
            pltpu.VMEM((n_hops, CHUNK, f), jnp.float32),
            pltpu.SemaphoreType.DMA((n_hops,)),
            pltpu.SemaphoreType.DMA((n_hops,)),
        ],
        compiler_params=pltpu.CompilerParams(
            collective_id=0,
            vmem_limit_bytes=100 * 1024 * 1024,
        ),
    )(x, dy)


# device time: 156788 ns/iter; 1.4092x vs baseline; 1.4092x over previous
import jax
import jax.numpy as jnp
from jax import lax
from jax.experimental import pallas as pl
from jax.experimental.pallas import tpu as pltpu

N_DEV = 16
CHUNK = 64


def kernel(x, dy):
    k, m = x.shape
    _, f = dy.shape
    n_hops = N_DEV - 1
    fh = f // 2

    def body(x_ref, dy_ref, out_ref, acc_ref,
             comm_cw, comm_ccw, send_cw, recv_cw, send_ccw, recv_ccw):
        me = lax.axis_index("i")
        left = lax.rem(me + N_DEV - 1, N_DEV)
        right = lax.rem(me + 1, N_DEV)

        barrier_sem = pltpu.get_barrier_semaphore()
        for nbr in (left, right):
            pl.semaphore_signal(
                barrier_sem, inc=1,
                device_id=(nbr,), device_id_type=pl.DeviceIdType.MESH,
            )
        pl.semaphore_wait(barrier_sem, 2)

        acc_ref[:, :] = lax.dot_general(
            x_ref[:, :], dy_ref[:, :],
            dimension_numbers=(((0,), (0,)), ((), ())),
            preferred_element_type=jnp.float32,
        )

        for h in range(n_hops):
            c_scw = lax.rem(me + 2 * N_DEV - 1 - h, N_DEV)
            rdma_cw = pltpu.make_async_remote_copy(
                src_ref=acc_ref.at[pl.ds(c_scw * CHUNK, CHUNK), pl.ds(0, fh)],
                dst_ref=comm_cw.at[h],
                send_sem=send_cw.at[h],
                recv_sem=recv_cw.at[h],
                device_id=(right,),
                device_id_type=pl.DeviceIdType.MESH,
            )
            rdma_cw.start()
            c_sccw = lax.rem(me + 1 + h, N_DEV)
            rdma_ccw = pltpu.make_async_remote_copy(
                src_ref=acc_ref.at[pl.ds(c_sccw * CHUNK, CHUNK), pl.ds(fh, fh)],
                dst_ref=comm_ccw.at[h],
                send_sem=send_ccw.at[h],
                recv_sem=recv_ccw.at[h],
                device_id=(left,),
                device_id_type=pl.DeviceIdType.MESH,
            )
            rdma_ccw.start()

            rdma_cw.wait()
            c_rcw = lax.rem(me + 2 * N_DEV - 2 - h, N_DEV)
            sl = pl.ds(c_rcw * CHUNK, CHUNK)
            acc_ref[sl, pl.ds(0, fh)] = acc_ref[sl, pl.ds(0, fh)] + comm_cw[h]

            rdma_ccw.wait()
            c_rccw = lax.rem(me + 2 + h, N_DEV)
            sr = pl.ds(c_rccw * CHUNK, CHUNK)
            acc_ref[sr, pl.ds(fh, fh)] = acc_ref[sr, pl.ds(fh, fh)] + comm_ccw[h]

        out_ref[:, :] = acc_ref[pl.ds(me * CHUNK, CHUNK), :]

    return pl.pallas_call(
        body,
        out_shape=jax.ShapeDtypeStruct((CHUNK, f), jnp.float32),
        in_specs=[
            pl.BlockSpec(memory_space=pltpu.VMEM),
            pl.BlockSpec(memory_space=pltpu.VMEM),
        ],
        out_specs=pl.BlockSpec(memory_space=pltpu.VMEM),
        scratch_shapes=[
            pltpu.VMEM((N_DEV * CHUNK, f), jnp.float32),
            pltpu.VMEM((n_hops, CHUNK, fh), jnp.float32),
            pltpu.VMEM((n_hops, CHUNK, fh), jnp.float32),
            pltpu.SemaphoreType.DMA((n_hops,)),
            pltpu.SemaphoreType.DMA((n_hops,)),
            pltpu.SemaphoreType.DMA((n_hops,)),
            pltpu.SemaphoreType.DMA((n_hops,)),
        ],
        compiler_params=pltpu.CompilerParams(
            collective_id=0,
            vmem_limit_bytes=100 * 1024 * 1024,
        ),
    )(x, dy)


# device time: 116168 ns/iter; 1.9019x vs baseline; 1.3497x over previous
import jax
import jax.numpy as jnp
from jax import lax
from jax.experimental import pallas as pl
from jax.experimental.pallas import tpu as pltpu

N_DEV = 16
CHUNK = 64
T = 2


def kernel(x, dy):
    k, m = x.shape
    _, f = dy.shape
    n_hops = N_DEV - 1
    fh = f // 2
    w = fh // T

    def body(x_ref, dy_ref, out_ref, acc_ref,
             comm_cw, comm_ccw, send_cw, recv_cw, send_ccw, recv_ccw):
        me = lax.axis_index("i")
        left = lax.rem(me + N_DEV - 1, N_DEV)
        right = lax.rem(me + 1, N_DEV)

        barrier_sem = pltpu.get_barrier_semaphore()
        for nbr in (left, right):
            pl.semaphore_signal(
                barrier_sem, inc=1,
                device_id=(nbr,), device_id_type=pl.DeviceIdType.MESH,
            )
        pl.semaphore_wait(barrier_sem, 2)

        acc_ref[:, :] = lax.dot_general(
            x_ref[:, :], dy_ref[:, :],
            dimension_numbers=(((0,), (0,)), ((), ())),
            preferred_element_type=jnp.float32,
        )

        def make(cw: bool, t: int, h: int):
            if cw:
                c_send = lax.rem(me + 2 * N_DEV - 1 - h, N_DEV)
                cols = pl.ds(t * w, w)
                return pltpu.make_async_remote_copy(
                    src_ref=acc_ref.at[pl.ds(c_send * CHUNK, CHUNK), cols],
                    dst_ref=comm_cw.at[h, :, pl.ds(t * w, w)],
                    send_sem=send_cw.at[h, t],
                    recv_sem=recv_cw.at[h, t],
                    device_id=(right,),
                    device_id_type=pl.DeviceIdType.MESH,
                )
            c_send = lax.rem(me + 1 + h, N_DEV)
            cols = pl.ds(fh + t * w, w)
            return pltpu.make_async_remote_copy(
                src_ref=acc_ref.at[pl.ds(c_send * CHUNK, CHUNK), cols],
                dst_ref=comm_ccw.at[h, :, pl.ds(t * w, w)],
                send_sem=send_ccw.at[h, t],
                recv_sem=recv_ccw.at[h, t],
                device_id=(left,),
                device_id_type=pl.DeviceIdType.MESH,
            )

        units = [(True, t) for t in range(T)] + [(False, t) for t in range(T)]

        for cw, t in units:
            make(cw, t, 0).start()

        for h in range(n_hops):
            c_rcw = lax.rem(me + 2 * N_DEV - 2 - h, N_DEV)
            c_rccw = lax.rem(me + 2 + h, N_DEV)
            for cw, t in units:
                r = make(cw, t, h)
                r.wait_recv()
                if cw:
                    rows = pl.ds(c_rcw * CHUNK, CHUNK)
                    cols = pl.ds(t * w, w)
                    acc_ref[rows, cols] = acc_ref[rows, cols] + comm_cw[
                        h, :, t * w:(t + 1) * w]
                else:
                    rows = pl.ds(c_rccw * CHUNK, CHUNK)
                    cols = pl.ds(fh + t * w, w)
                    acc_ref[rows, cols] = acc_ref[rows, cols] + comm_ccw[
                        h, :, t * w:(t + 1) * w]
                if h + 1 < n_hops:
                    make(cw, t, h + 1).start()
                r.wait_send()

        out_ref[:, :] = acc_ref[pl.ds(me * CHUNK, CHUNK), :]

    return pl.pallas_call(
        body,
        out_shape=jax.ShapeDtypeStruct((CHUNK, f), jnp.float32),
        in_specs=[
            pl.BlockSpec(memory_space=pltpu.VMEM),
            pl.BlockSpec(memory_space=pltpu.VMEM),
        ],
        out_specs=pl.BlockSpec(memory_space=pltpu.VMEM),
        scratch_shapes=[
            pltpu.VMEM((N_DEV * CHUNK, f), jnp.float32),
            pltpu.VMEM((n_hops, CHUNK, fh), jnp.float32),
            pltpu.VMEM((n_hops, CHUNK, fh), jnp.float32),
            pltpu.SemaphoreType.DMA((n_hops, T)),
            pltpu.SemaphoreType.DMA((n_hops, T)),
            pltpu.SemaphoreType.DMA((n_hops, T)),
            pltpu.SemaphoreType.DMA((n_hops, T)),
        ],
        compiler_params=pltpu.CompilerParams(
            collective_id=0,
            vmem_limit_bytes=100 * 1024 * 1024,
        ),
    )(x, dy)
